# baseline (speedup 1.0000x reference)
import jax
import jax.numpy as jnp
from jax import lax
from jax.experimental import pallas as pl
from jax.experimental.pallas import tpu as pltpu
from jax.experimental.pallas import tpu_sc as plsc

NUM_FIELDS = 26
VOCAB = 10000
D_EMBED = 100
BATCH = 16384
NCOLS = NUM_FIELDS * D_EMBED

_INFO = plsc.get_sparse_core_info()
NC = _INFO.num_cores
NS = _INFO.num_subcores
NW = NC * NS
L = _INFO.num_lanes

COLS_BASE = NCOLS // NW
COLS_EXTRA = NCOLS % NW
NGROUPS = BATCH // L


def _embed_body(xt_hbm, tt_hbm, out_hbm, xf_v, col_v, ocol_v, sem):
    wid = lax.axis_index("s") * NC + lax.axis_index("c")
    c0 = wid * COLS_BASE + lax.min(wid, COLS_EXTRA)
    cnt = COLS_BASE + jnp.where(wid < COLS_EXTRA, 1, 0)
    c1 = c0 + cnt
    f0 = c0 // D_EMBED
    f1 = (c1 - 1) // D_EMBED

    def field_body(f, carry):
        pltpu.sync_copy(xt_hbm.at[f], xf_v)
        dlo = lax.max(c0, f * D_EMBED) - f * D_EMBED
        dhi = lax.min(c1, (f + 1) * D_EMBED) - f * D_EMBED

        def col_body(d, carry2):
            pltpu.sync_copy(tt_hbm.at[f, d], col_v)

            def grp_body(k, carry3):
                idx = xf_v[pl.ds(k * L, L)]
                ocol_v[pl.ds(k * L, L)] = plsc.load_gather(col_v, [idx])
                return carry3

            lax.fori_loop(0, NGROUPS, grp_body, 0, unroll=8)
            pltpu.sync_copy(ocol_v, out_hbm.at[f * D_EMBED + d])
            return carry2

        lax.fori_loop(dlo, dhi, col_body, 0)
        return carry

    lax.fori_loop(f0, f1 + 1, field_body, 0)


@jax.jit
def _embed(x_t, tab_t):
    mesh = plsc.VectorSubcoreMesh(core_axis_name="c", subcore_axis_name="s")
    call = pl.kernel(
        _embed_body,
        out_type=jax.ShapeDtypeStruct((NCOLS, BATCH), jnp.float32),
        mesh=mesh,
        scratch_types=[
            pltpu.VMEM((BATCH,), jnp.int32),
            pltpu.VMEM((VOCAB,), jnp.float32),
            pltpu.VMEM((BATCH,), jnp.float32),
            pltpu.SemaphoreType.DMA,
        ],
        compiler_params=pltpu.CompilerParams(
            use_tc_tiling_on_sc=False, needs_layout_passes=False),
    )
    return call(x_t, tab_t)


def kernel(x_cat, tables):
    x_t = x_cat.T
    tab_t = tables.transpose(0, 2, 1)
    out_t = _embed(x_t, tab_t)
    return out_t.T


# speedup vs baseline: 1.9658x; 1.9658x over previous
import jax
import jax.numpy as jnp
from jax import lax
from jax.experimental import pallas as pl
from jax.experimental.pallas import tpu as pltpu
from jax.experimental.pallas import tpu_sc as plsc

NUM_FIELDS = 26
VOCAB = 10000
D_EMBED = 100
BATCH = 16384
D_PAD = 128

_INFO = plsc.get_sparse_core_info()
NC = _INFO.num_cores
NS = _INFO.num_subcores
NW = NC * NS
L = _INFO.num_lanes

N_ROWS = BATCH * NUM_FIELDS
ROWS_PER_W = N_ROWS // NW
CHUNK = 128
NCHUNKS = ROWS_PER_W // CHUNK
NBUF = 2


def _gather_body(idx_hbm, tab_hbm, out_hbm, idx_all, rows_v, gsem, wsem):
    wid = lax.axis_index("s") * NC + lax.axis_index("c")
    wbase = wid * ROWS_PER_W
    pltpu.sync_copy(idx_hbm.at[pl.ds(wbase, ROWS_PER_W)], idx_all)

    def out_slice(c):
        return out_hbm.at[pl.ds(wbase + c * CHUNK, CHUNK)]

    def pair_body(i, carry):
        c0 = i * NBUF

        @pl.when(i > 0)
        def _():
            for b in range(NBUF):
                pltpu.make_async_copy(rows_v.at[b], out_slice(0), wsem[b]).wait()

        for b in range(NBUF):
            pltpu.async_copy(
                tab_hbm.at[idx_all.at[pl.ds((c0 + b) * CHUNK, CHUNK)]],
                rows_v.at[b], gsem[b])
        for b in range(NBUF):
            pltpu.make_async_copy(
                tab_hbm.at[idx_all.at[pl.ds((c0 + b) * CHUNK, CHUNK)]],
                rows_v.at[b], gsem[b]).wait()
            pltpu.async_copy(rows_v.at[b], out_slice(c0 + b), wsem[b])
        return carry

    lax.fori_loop(0, NCHUNKS // NBUF, pair_body, 0)
    for b in range(NBUF):
        pltpu.make_async_copy(rows_v.at[b], out_slice(0), wsem[b]).wait()


@jax.jit
def _gather(x_flat, flat_table):
    mesh = plsc.VectorSubcoreMesh(core_axis_name="c", subcore_axis_name="s")
    call = pl.kernel(
        _gather_body,
        out_type=jax.ShapeDtypeStruct((N_ROWS, D_PAD), jnp.float32),
        mesh=mesh,
        scratch_types=[
            pltpu.VMEM((ROWS_PER_W,), jnp.int32),
            pltpu.VMEM((NBUF, CHUNK, D_PAD), jnp.float32),
            [pltpu.SemaphoreType.DMA] * NBUF,
            [pltpu.SemaphoreType.DMA] * NBUF,
        ],
        compiler_params=pltpu.CompilerParams(
            use_tc_tiling_on_sc=False, needs_layout_passes=False),
    )
    return call(x_flat, flat_table)


def kernel(x_cat, tables):
    x_flat = (x_cat + jnp.arange(NUM_FIELDS, dtype=jnp.int32) * VOCAB).reshape(N_ROWS)
    flat_table = lax.pad(
        tables, jnp.float32(0), ((0, 0, 0), (0, 0, 0), (0, D_PAD - D_EMBED, 0))
    ).reshape(NUM_FIELDS * VOCAB, D_PAD)
    out = _gather(x_flat, flat_table)
    return out[:, :D_EMBED].reshape(BATCH, NUM_FIELDS * D_EMBED)
